# baseline (speedup 1.0000x reference)
import functools

import jax
import jax.numpy as jnp
from jax import lax
from jax.experimental import pallas as pl
from jax.experimental.pallas import tpu as pltpu
from jax.experimental.pallas import tpu_sc as plsc

B = 1024
L = 20
DIM = 100
ALPHA = 0.2
BB = 8
R = BB * L
GR = B // BB
DIMP = 128
NW = 32
CHUNK = 128


def _sc_gather(table, idx):
    n = idx.shape[0]
    d = table.shape[1]
    per_w = n // NW
    nchunks = per_w // CHUNK
    mesh = plsc.VectorSubcoreMesh(core_axis_name="c", subcore_axis_name="s")

    @functools.partial(
        pl.kernel,
        out_type=jax.ShapeDtypeStruct((n, d), jnp.float32),
        mesh=mesh,
        scratch_types=[
            pltpu.VMEM((per_w,), jnp.int32),
            pltpu.VMEM((CHUNK, d), jnp.float32),
            pltpu.SemaphoreType.DMA,
        ],
        name="sc_gather3",
    )
    def k(table_hbm, idx_hbm, out_hbm, idx_v, rows_v, sem):
        wid = lax.axis_index("s") * 2 + lax.axis_index("c")
        base = wid * per_w
        pltpu.sync_copy(idx_hbm.at[pl.ds(base, per_w)], idx_v)

        def body(i, carry):
            off = i * CHUNK
            pltpu.async_copy(
                table_hbm.at[idx_v.at[pl.ds(off, CHUNK)]], rows_v, sem
            ).wait()
            pltpu.sync_copy(rows_v, out_hbm.at[pl.ds(base + off, CHUNK)])
            return carry

        lax.fori_loop(0, nchunks, body, 0)

    return k(table, idx)


def _pad_rows(emb):
    rows = emb.shape[0]
    blk = 2000

    def body(s_ref, d_ref):
        d_ref[:, :DIM] = s_ref[...]
        d_ref[:, DIM:] = jnp.zeros((blk, DIMP - DIM), jnp.float32)

    return pl.pallas_call(
        body,
        grid=rows // blk,
        in_specs=[pl.BlockSpec((blk, DIM), lambda g: (g, 0))],
        out_specs=pl.BlockSpec((blk, DIMP), lambda g: (g, 0)),
        out_shape=jax.ShapeDtypeStruct((rows, DIMP), jnp.float32),
        compiler_params=pltpu.CompilerParams(
            dimension_semantics=("arbitrary",),
        ),
    )(emb)


def _leaky(x):
    return jnp.where(x >= 0, x, ALPHA * x)


def _tc_body(h1_ref, h2_ref, hm_ref, adjt_ref, tadjt_ref, ain_ref, aout_ref,
             la1_ref, mix_ref, weio_ref, win_ref, wout_ref, whh_ref,
             beio_ref, biaoh_ref, bih_ref, bhh_ref,
             o1_ref, o2_ref, om_ref):
    big_neg = jnp.float32(-9e15)

    def local_agg(h, adjt, a_ref, o_ref):
        alpha = jnp.full((R, R), big_neg, jnp.float32)
        for k in range(4):
            g = lax.dot_general(h * a_ref[k:k + 1, :], h, (((1,), (1,)), ((), ())),
                                preferred_element_type=jnp.float32)
            alpha = jnp.where(adjt == k + 1, _leaky(g), alpha)
        alpha = alpha - jnp.max(alpha, axis=1, keepdims=True)
        p = jnp.exp(alpha)
        alpha = p * (1.0 / jnp.sum(p, axis=1, keepdims=True))
        o = jnp.dot(alpha, h, preferred_element_type=jnp.float32)
        o_ref[...] = o[:, :DIM].reshape(BB, L, DIM)

    h1 = h1_ref[...]
    h2 = h2_ref[...]
    hm = hm_ref[...]

    local_agg(h1, adjt_ref[0].astype(jnp.int32), la1_ref, o1_ref)
    local_agg(hm, tadjt_ref[0].astype(jnp.int32), mix_ref, om_ref)

    x_all = jnp.dot(h2, weio_ref[...], preferred_element_type=jnp.float32) + beio_ref[...]
    input_in = jnp.dot(ain_ref[0], x_all[:, :DIMP],
                       preferred_element_type=jnp.float32) + biaoh_ref[:, :DIMP]
    input_out = jnp.dot(aout_ref[0], x_all[:, DIMP:],
                        preferred_element_type=jnp.float32) + biaoh_ref[:, DIMP:]
    gi = (jnp.dot(input_in, win_ref[...], preferred_element_type=jnp.float32)
          + jnp.dot(input_out, wout_ref[...], preferred_element_type=jnp.float32)
          + bih_ref[...])
    gh = jnp.dot(h2, whh_ref[...], preferred_element_type=jnp.float32) + bhh_ref[...]
    s = DIMP
    resetgate = jax.nn.sigmoid(gi[:, :DIM] + gh[:, :DIM])
    inputgate = jax.nn.sigmoid(gi[:, s:s + DIM] + gh[:, s:s + DIM])
    newgate = jnp.tanh(gi[:, 2 * s:2 * s + DIM] + resetgate * gh[:, 2 * s:2 * s + DIM])
    h2s = h2[:, :DIM]
    o2_ref[...] = (newgate + inputgate * (newgate - h2s)).reshape(BB, L, DIM)


def _tc_compute(rows, adjt, tadjt, ain, aout, weights):
    nb = (B * L) // R
    full = lambda s: pl.BlockSpec(s, lambda g: tuple(0 for _ in s))
    bd3 = lambda: pl.BlockSpec((1, R, R), lambda g: (g, 0, 0))
    in_specs = [
        pl.BlockSpec((R, DIMP), lambda g: (g, 0)),
        pl.BlockSpec((R, DIMP), lambda g: (g + nb, 0)),
        pl.BlockSpec((R, DIMP), lambda g: (g + 2 * nb, 0)),
        bd3(), bd3(), bd3(), bd3(),
        full((4, DIMP)), full((4, DIMP)),
        full((DIMP, 2 * DIMP)), full((DIMP, 3 * DIMP)), full((DIMP, 3 * DIMP)),
        full((DIMP, 3 * DIMP)),
        full((1, 2 * DIMP)), full((1, 2 * DIMP)), full((1, 3 * DIMP)),
        full((1, 3 * DIMP)),
    ]
    o_spec = pl.BlockSpec((BB, L, DIM), lambda g: (g, 0, 0))
    out_shape = tuple(jax.ShapeDtypeStruct((B, L, DIM), jnp.float32) for _ in range(3))
    return pl.pallas_call(
        _tc_body,
        grid=GR,
        in_specs=in_specs,
        out_specs=(o_spec, o_spec, o_spec),
        out_shape=out_shape,
        compiler_params=pltpu.CompilerParams(
            dimension_semantics=("arbitrary",),
        ),
    )(rows, rows, rows, adjt, tadjt, ain, aout, *weights)


def _block_diag_expand(x4, dtype):
    eye = (jnp.arange(BB)[:, None] == jnp.arange(BB)[None, :])
    y = jnp.where(eye[None, :, None, :, None], x4[:, :, :, None, :].astype(dtype),
                  jnp.zeros((), dtype))
    return y.reshape(GR, R, R)


def _stack_lanes(ws, width):
    out = jnp.zeros((DIMP, width), jnp.float32)
    for i, w in enumerate(ws):
        out = out.at[:DIM, i * DIMP:i * DIMP + DIM].set(w)
    return out


def _stack_bias(bs, width):
    out = jnp.zeros((1, width), jnp.float32)
    for i, b in enumerate(bs):
        out = out.at[0, i * DIMP:i * DIMP + DIM].set(b)
    return out


def kernel(inputs, adj, mask_item, item, items_ID, adj_ID, total_items, total_adj,
           embedding, la1_a, mix_a, Wei, bei, Weo, beo, w_ih, w_hh, b_ih, b_hh,
           b_iah, b_oah):
    n = B * L
    idx_all = jnp.concatenate([
        inputs.reshape(-1), items_ID.reshape(-1), total_items.reshape(-1)
    ]).astype(jnp.int32)
    emb_p = _pad_rows(embedding)
    rows = _sc_gather(emb_p, idx_all)

    adjt = _block_diag_expand(adj.astype(jnp.int8).reshape(GR, BB, L, L), jnp.int8)
    tadjt = _block_diag_expand(total_adj.astype(jnp.int8).reshape(GR, BB, L, L),
                               jnp.int8)
    aid = adj_ID.reshape(GR, BB, L, 2 * L)
    ain = _block_diag_expand(aid[..., :L], jnp.float32)
    aout = _block_diag_expand(aid[..., L:], jnp.float32)

    pad4 = lambda w: jnp.pad(w.T, ((0, 0), (0, DIMP - DIM)))
    weights = (
        pad4(la1_a), pad4(mix_a),
        _stack_lanes([Wei.T, Weo.T], 2 * DIMP),
        _stack_lanes([w_ih[0:DIM, 0:DIM].T, w_ih[DIM:2 * DIM, 0:DIM].T,
                      w_ih[2 * DIM:, 0:DIM].T], 3 * DIMP),
        _stack_lanes([w_ih[0:DIM, DIM:].T, w_ih[DIM:2 * DIM, DIM:].T,
                      w_ih[2 * DIM:, DIM:].T], 3 * DIMP),
        _stack_lanes([w_hh[0:DIM].T, w_hh[DIM:2 * DIM].T, w_hh[2 * DIM:].T],
                     3 * DIMP),
        _stack_bias([bei, beo], 2 * DIMP),
        _stack_bias([b_iah, b_oah], 2 * DIMP),
        _stack_bias([b_ih[0:DIM], b_ih[DIM:2 * DIM], b_ih[2 * DIM:]], 3 * DIMP),
        _stack_bias([b_hh[0:DIM], b_hh[DIM:2 * DIM], b_hh[2 * DIM:]], 3 * DIMP),
    )
    return _tc_compute(rows, adjt, tadjt, ain, aout, weights)


# speedup vs baseline: 2.0541x; 2.0541x over previous
import functools

import jax
import jax.numpy as jnp
from jax import lax
from jax.experimental import pallas as pl
from jax.experimental.pallas import tpu as pltpu
from jax.experimental.pallas import tpu_sc as plsc

B = 1024
L = 20
DIM = 100
ALPHA = 0.2
BB = 64
R = BB * L
GR = B // BB
DIMP = 128
NW = 32
CHUNK = 128


def _sc_gather(table, idx):
    n = idx.shape[0]
    d = table.shape[1]
    per_w = n // NW
    nchunks = per_w // CHUNK
    mesh = plsc.VectorSubcoreMesh(core_axis_name="c", subcore_axis_name="s")

    @functools.partial(
        pl.kernel,
        out_type=jax.ShapeDtypeStruct((n, d), jnp.float32),
        mesh=mesh,
        scratch_types=[
            pltpu.VMEM((per_w,), jnp.int32),
            pltpu.VMEM((CHUNK, d), jnp.float32),
            pltpu.SemaphoreType.DMA,
        ],
        name="sc_gather3",
    )
    def k(table_hbm, idx_hbm, out_hbm, idx_v, rows_v, sem):
        wid = lax.axis_index("s") * 2 + lax.axis_index("c")
        base = wid * per_w
        pltpu.sync_copy(idx_hbm.at[pl.ds(base, per_w)], idx_v)

        def body(i, carry):
            off = i * CHUNK
            pltpu.async_copy(
                table_hbm.at[idx_v.at[pl.ds(off, CHUNK)]], rows_v, sem
            ).wait()
            pltpu.sync_copy(rows_v, out_hbm.at[pl.ds(base + off, CHUNK)])
            return carry

        lax.fori_loop(0, nchunks, body, 0)

    return k(table, idx)


def _pad_rows(emb):
    rows = emb.shape[0]
    blk = 2000

    def body(s_ref, d_ref):
        d_ref[:, :DIM] = s_ref[...]
        d_ref[:, DIM:] = jnp.zeros((blk, DIMP - DIM), jnp.float32)

    return pl.pallas_call(
        body,
        grid=rows // blk,
        in_specs=[pl.BlockSpec((blk, DIM), lambda g: (g, 0))],
        out_specs=pl.BlockSpec((blk, DIMP), lambda g: (g, 0)),
        out_shape=jax.ShapeDtypeStruct((rows, DIMP), jnp.float32),
        compiler_params=pltpu.CompilerParams(
            dimension_semantics=("arbitrary",),
        ),
    )(emb)


def _leaky(x):
    return jnp.where(x >= 0, x, ALPHA * x)


def _tc_body(h1_ref, h2_ref, hm_ref, adj_ref, tadj_ref, adjid_ref,
             la1_ref, mix_ref, weio_ref, win_ref, wout_ref, whh_ref,
             beio_ref, biaoh_ref, bih_ref, bhh_ref,
             o1_ref, o2_ref, om_ref):
    big_neg = jnp.float32(-9e15)

    def local_agg(h, adjt, a_ref, o_ref):
        h3 = h.reshape(BB, L, DIMP)
        alpha = jnp.full((BB, L, L), big_neg, jnp.float32)
        for k in range(4):
            hw3 = (h * a_ref[k:k + 1, :]).reshape(BB, L, DIMP)
            g = lax.dot_general(hw3, h3, (((2,), (2,)), ((0,), (0,))),
                                preferred_element_type=jnp.float32)
            alpha = jnp.where(adjt == k + 1, _leaky(g), alpha)
        alpha = alpha - jnp.max(alpha, axis=2, keepdims=True)
        p = jnp.exp(alpha)
        alpha = p * (1.0 / jnp.sum(p, axis=2, keepdims=True))
        o = lax.dot_general(alpha, h3, (((2,), (1,)), ((0,), (0,))),
                            preferred_element_type=jnp.float32)
        o_ref[...] = o[:, :, :DIM]

    h1 = h1_ref[...]
    h2 = h2_ref[...]
    hm = hm_ref[...]

    local_agg(h1, adj_ref[...], la1_ref, o1_ref)
    local_agg(hm, tadj_ref[...], mix_ref, om_ref)

    adjid = adjid_ref[...]
    x_all = jnp.dot(h2, weio_ref[...], preferred_element_type=jnp.float32) + beio_ref[...]
    bdot = lambda a, x: lax.dot_general(a, x, (((2,), (1,)), ((0,), (0,))),
                                        preferred_element_type=jnp.float32)
    x_in3 = x_all[:, :DIMP].reshape(BB, L, DIMP)
    x_out3 = x_all[:, DIMP:].reshape(BB, L, DIMP)
    input_in = bdot(adjid[:, :, :L], x_in3).reshape(R, DIMP) + biaoh_ref[:, :DIMP]
    input_out = bdot(adjid[:, :, L:], x_out3).reshape(R, DIMP) + biaoh_ref[:, DIMP:]
    gi = (jnp.dot(input_in, win_ref[...], preferred_element_type=jnp.float32)
          + jnp.dot(input_out, wout_ref[...], preferred_element_type=jnp.float32)
          + bih_ref[...])
    gh = jnp.dot(h2, whh_ref[...], preferred_element_type=jnp.float32) + bhh_ref[...]
    s = DIMP
    resetgate = jax.nn.sigmoid(gi[:, :DIM] + gh[:, :DIM])
    inputgate = jax.nn.sigmoid(gi[:, s:s + DIM] + gh[:, s:s + DIM])
    newgate = jnp.tanh(gi[:, 2 * s:2 * s + DIM] + resetgate * gh[:, 2 * s:2 * s + DIM])
    h2s = h2[:, :DIM]
    o2_ref[...] = (newgate + inputgate * (newgate - h2s)).reshape(BB, L, DIM)


def _tc_compute(rows, adj, tadj, adj_ID, weights):
    nb = (B * L) // R
    full = lambda s: pl.BlockSpec(s, lambda g: tuple(0 for _ in s))
    in_specs = [
        pl.BlockSpec((R, DIMP), lambda g: (g, 0)),
        pl.BlockSpec((R, DIMP), lambda g: (g + nb, 0)),
        pl.BlockSpec((R, DIMP), lambda g: (g + 2 * nb, 0)),
        pl.BlockSpec((BB, L, L), lambda g: (g, 0, 0)),
        pl.BlockSpec((BB, L, L), lambda g: (g, 0, 0)),
        pl.BlockSpec((BB, L, 2 * L), lambda g: (g, 0, 0)),
        full((4, DIMP)), full((4, DIMP)),
        full((DIMP, 2 * DIMP)), full((DIMP, 3 * DIMP)), full((DIMP, 3 * DIMP)),
        full((DIMP, 3 * DIMP)),
        full((1, 2 * DIMP)), full((1, 2 * DIMP)), full((1, 3 * DIMP)),
        full((1, 3 * DIMP)),
    ]
    o_spec = pl.BlockSpec((BB, L, DIM), lambda g: (g, 0, 0))
    out_shape = tuple(jax.ShapeDtypeStruct((B, L, DIM), jnp.float32) for _ in range(3))
    return pl.pallas_call(
        _tc_body,
        grid=GR,
        in_specs=in_specs,
        out_specs=(o_spec, o_spec, o_spec),
        out_shape=out_shape,
        compiler_params=pltpu.CompilerParams(
            dimension_semantics=("arbitrary",),
        ),
    )(rows, rows, rows, adj, tadj, adj_ID, *weights)


def _stack_lanes(ws, width):
    out = jnp.zeros((DIMP, width), jnp.float32)
    for i, w in enumerate(ws):
        out = out.at[:DIM, i * DIMP:i * DIMP + DIM].set(w)
    return out


def _stack_bias(bs, width):
    out = jnp.zeros((1, width), jnp.float32)
    for i, b in enumerate(bs):
        out = out.at[0, i * DIMP:i * DIMP + DIM].set(b)
    return out


def kernel(inputs, adj, mask_item, item, items_ID, adj_ID, total_items, total_adj,
           embedding, la1_a, mix_a, Wei, bei, Weo, beo, w_ih, w_hh, b_ih, b_hh,
           b_iah, b_oah):
    n = B * L
    idx_all = jnp.concatenate([
        inputs.reshape(-1), items_ID.reshape(-1), total_items.reshape(-1)
    ]).astype(jnp.int32)
    emb_p = _pad_rows(embedding)
    rows = _sc_gather(emb_p, idx_all)

    pad4 = lambda w: jnp.pad(w.T, ((0, 0), (0, DIMP - DIM)))
    weights = (
        pad4(la1_a), pad4(mix_a),
        _stack_lanes([Wei.T, Weo.T], 2 * DIMP),
        _stack_lanes([w_ih[0:DIM, 0:DIM].T, w_ih[DIM:2 * DIM, 0:DIM].T,
                      w_ih[2 * DIM:, 0:DIM].T], 3 * DIMP),
        _stack_lanes([w_ih[0:DIM, DIM:].T, w_ih[DIM:2 * DIM, DIM:].T,
                      w_ih[2 * DIM:, DIM:].T], 3 * DIMP),
        _stack_lanes([w_hh[0:DIM].T, w_hh[DIM:2 * DIM].T, w_hh[2 * DIM:].T],
                     3 * DIMP),
        _stack_bias([bei, beo], 2 * DIMP),
        _stack_bias([b_iah, b_oah], 2 * DIMP),
        _stack_bias([b_ih[0:DIM], b_ih[DIM:2 * DIM], b_ih[2 * DIM:]], 3 * DIMP),
        _stack_bias([b_hh[0:DIM], b_hh[DIM:2 * DIM], b_hh[2 * DIM:]], 3 * DIMP),
    )
    return _tc_compute(rows, adj.astype(jnp.int32), total_adj.astype(jnp.int32),
                       adj_ID, weights)
